# baseline (speedup 1.0000x reference)
import functools
import math

import jax
import jax.numpy as jnp
from jax import lax
from jax.experimental import pallas as pl
from jax.experimental.pallas import tpu as pltpu


def _fff_block_kernel(xh_ref, xl_ref, w1h_ref, w1l_ref, w2_ref, out_ref, *,
                      depth, n_pad):
    xh = xh_ref[...]
    xl = xl_ref[...]
    m = xh.shape[0]
    dn = (((1,), (1,)), ((), ()))
    f32 = jnp.float32
    w1h = w1h_ref[...]
    w1l = w1l_ref[...]
    logits = lax.dot_general(xh, w1h, dn, preferred_element_type=f32)
    logits += lax.dot_general(xl, w1h, dn, preferred_element_type=f32)
    logits += lax.dot_general(xh, w1l, dn, preferred_element_type=f32)
    logits += lax.dot_general(xl, w1l, dn, preferred_element_type=f32)

    p = jnp.zeros((m, 1), jnp.int32)
    pieces = []
    for lvl in range(depth):
        w = 1 << lvl
        base = w - 1
        sl = lax.slice(logits, (0, base), (m, base + w))
        col = lax.broadcasted_iota(jnp.int32, (m, w), 1)
        sel = col == p
        logit = jnp.sum(jnp.where(sel, sl, 0.0), axis=1, keepdims=True)
        act = jax.nn.gelu(logit)
        pieces.append(jnp.where(sel, act, 0.0))
        p = 2 * p + (logit > 0.0).astype(jnp.int32)
    n_nodes = (1 << depth) - 1
    if n_pad > n_nodes:
        pieces.append(jnp.zeros((m, n_pad - n_nodes), f32))
    acts = jnp.concatenate(pieces, axis=1).astype(jnp.bfloat16)

    out_ref[...] = lax.dot_general(
        acts, w2_ref[...], (((1,), (0,)), ((), ())),
        preferred_element_type=f32,
    )


def _split_bf16(a):
    bits = lax.bitcast_convert_type(a, jnp.uint32)
    rounded = (bits + jnp.uint32(0x7FFF) + ((bits >> 16) & jnp.uint32(1))) \
        & jnp.uint32(0xFFFF0000)
    hi = lax.bitcast_convert_type(rounded, jnp.float32)
    lo = a - hi
    return hi.astype(jnp.bfloat16), lo.astype(jnp.bfloat16)


@jax.jit
def kernel(input, w1s, w2s):
    tokens, nin = input.shape
    n_nodes, nout = w2s.shape
    depth = int(math.log2(n_nodes + 1))
    n_pad = n_nodes + 1

    w1p = jnp.concatenate([w1s, jnp.zeros((n_pad - n_nodes, nin), w1s.dtype)])
    w1h, w1l = _split_bf16(w1p)
    w2p = jnp.concatenate([w2s, jnp.zeros((n_pad - n_nodes, nout), w2s.dtype)])
    w2p = w2p.astype(jnp.bfloat16)
    xh, xl = _split_bf16(input)

    m = 512
    grid = (tokens // m,)
    return pl.pallas_call(
        functools.partial(_fff_block_kernel, depth=depth, n_pad=n_pad),
        grid=grid,
        in_specs=[
            pl.BlockSpec((m, nin), lambda i: (i, 0)),
            pl.BlockSpec((m, nin), lambda i: (i, 0)),
            pl.BlockSpec((n_pad, nin), lambda i: (0, 0)),
            pl.BlockSpec((n_pad, nin), lambda i: (0, 0)),
            pl.BlockSpec((n_pad, nout), lambda i: (0, 0)),
        ],
        out_specs=pl.BlockSpec((m, nout), lambda i: (i, 0)),
        out_shape=jax.ShapeDtypeStruct((tokens, nout), jnp.float32),
    )(xh, xl, w1h, w1l, w2p)


# speedup vs baseline: 1.1988x; 1.1988x over previous
import functools
import math

import jax
import jax.numpy as jnp
from jax import lax
from jax.experimental import pallas as pl
from jax.experimental.pallas import tpu as pltpu


def _fff_block_kernel(x_ref, w1h_ref, w1l_ref, w2_ref, out_ref, *,
                      depth, n_pad):
    x = x_ref[...]
    m = x.shape[0]
    bits = lax.bitcast_convert_type(x, jnp.uint32)
    rounded = (bits + jnp.uint32(0x7FFF) + ((bits >> 16) & jnp.uint32(1))) \
        & jnp.uint32(0xFFFF0000)
    hi = lax.bitcast_convert_type(rounded, jnp.float32)
    xh = hi.astype(jnp.bfloat16)
    xl = (x - hi).astype(jnp.bfloat16)
    dn = (((1,), (1,)), ((), ()))
    f32 = jnp.float32
    w1h = w1h_ref[...]
    w1l = w1l_ref[...]
    logits = lax.dot_general(xh, w1h, dn, preferred_element_type=f32)
    logits += lax.dot_general(xl, w1h, dn, preferred_element_type=f32)
    logits += lax.dot_general(xh, w1l, dn, preferred_element_type=f32)
    logits += lax.dot_general(xl, w1l, dn, preferred_element_type=f32)

    p = jnp.zeros((m, 1), jnp.int32)
    pieces = []
    for lvl in range(depth):
        w = 1 << lvl
        base = w - 1
        sl = lax.slice(logits, (0, base), (m, base + w))
        col = lax.broadcasted_iota(jnp.int32, (m, w), 1)
        sel = col == p
        logit = jnp.sum(jnp.where(sel, sl, 0.0), axis=1, keepdims=True)
        act = jax.nn.gelu(logit)
        pieces.append(jnp.where(sel, act, 0.0))
        p = 2 * p + (logit > 0.0).astype(jnp.int32)
    n_nodes = (1 << depth) - 1
    if n_pad > n_nodes:
        pieces.append(jnp.zeros((m, n_pad - n_nodes), f32))
    acts = jnp.concatenate(pieces, axis=1).astype(jnp.bfloat16)

    out_ref[...] = lax.dot_general(
        acts, w2_ref[...], (((1,), (0,)), ((), ())),
        preferred_element_type=f32,
    )


def _split_bf16(a):
    bits = lax.bitcast_convert_type(a, jnp.uint32)
    rounded = (bits + jnp.uint32(0x7FFF) + ((bits >> 16) & jnp.uint32(1))) \
        & jnp.uint32(0xFFFF0000)
    hi = lax.bitcast_convert_type(rounded, jnp.float32)
    lo = a - hi
    return hi.astype(jnp.bfloat16), lo.astype(jnp.bfloat16)


@jax.jit
def kernel(input, w1s, w2s):
    tokens, nin = input.shape
    n_nodes, nout = w2s.shape
    depth = int(math.log2(n_nodes + 1))
    n_pad = n_nodes + 1

    w1p = jnp.concatenate([w1s, jnp.zeros((n_pad - n_nodes, nin), w1s.dtype)])
    w1h, w1l = _split_bf16(w1p)
    w2p = jnp.concatenate([w2s, jnp.zeros((n_pad - n_nodes, nout), w2s.dtype)])
    w2p = w2p.astype(jnp.bfloat16)

    m = 256
    grid = (tokens // m,)
    return pl.pallas_call(
        functools.partial(_fff_block_kernel, depth=depth, n_pad=n_pad),
        grid=grid,
        in_specs=[
            pl.BlockSpec((m, nin), lambda i: (i, 0)),
            pl.BlockSpec((n_pad, nin), lambda i: (0, 0)),
            pl.BlockSpec((n_pad, nin), lambda i: (0, 0)),
            pl.BlockSpec((n_pad, nout), lambda i: (0, 0)),
        ],
        out_specs=pl.BlockSpec((m, nout), lambda i: (i, 0)),
        out_shape=jax.ShapeDtypeStruct((tokens, nout), jnp.float32),
    )(input, w1h, w1l, w2p)
